# baseline (speedup 1.0000x reference)
import jax
import jax.numpy as jnp
from jax.experimental import pallas as pl
from jax.experimental.pallas import tpu as pltpu

BLK = 64
MAXBLK = 16


def _body(sid_smem, sid_row_ref, x_ref, w1_ref, b1_ref, w2_ref, b2_ref,
          out_ref, dest_ref, cnt_ref, bp_ref, bs_ref):
    s = pl.program_id(0)
    num_s = pl.num_programs(0)
    B = x_ref.shape[0]
    S = num_s

    @pl.when(s == 0)
    def _():
        for v in range(S):
            cnt_ref[v] = 0

        def hist(t, _):
            v = sid_smem[t]
            cnt_ref[v] = cnt_ref[v] + 1
            return 0

        jax.lax.fori_loop(0, B, hist, 0)
        running = 0
        for v in range(S):
            nb = (cnt_ref[v] + (BLK - 1)) // BLK
            bp_ref[v] = nb
            bs_ref[v] = running
            running = running + nb

        sid_row = sid_row_ref[...].astype(jnp.float32)
        iota_s = jax.lax.broadcasted_iota(jnp.int32, (128, 1), 0).astype(jnp.float32)
        onehot = (iota_s == sid_row).astype(jnp.float32)
        tri = (
            jax.lax.broadcasted_iota(jnp.int32, (B, B), 0)
            < jax.lax.broadcasted_iota(jnp.int32, (B, B), 1)
        ).astype(jnp.float32)
        prefix = jnp.dot(onehot, tri, preferred_element_type=jnp.float32)
        occ = jnp.sum(onehot * prefix, axis=0, keepdims=True)
        counts = jnp.sum(onehot, axis=1, keepdims=True)
        bp = jnp.floor((counts + (BLK - 1)) * (1.0 / BLK))
        tri_s = (
            jax.lax.broadcasted_iota(jnp.int32, (128, 128), 0)
            < jax.lax.broadcasted_iota(jnp.int32, (128, 128), 1)
        ).astype(jnp.float32)
        bstart = jax.lax.dot_general(
            tri_s, bp, (((0,), (0,)), ((), ())),
            preferred_element_type=jnp.float32,
        )
        bs_tok = jnp.sum(onehot * bstart, axis=0, keepdims=True)
        dest_ref[...] = bs_tok * BLK + occ

        out_ref[...] = jnp.zeros_like(out_ref)

    dest = dest_ref[...]
    x = x_ref[...]
    w1 = w1_ref[0]
    w2 = w2_ref[0]
    b1v = b1_ref[0]
    b2v = b2_ref[0]
    base = bs_ref[s]
    nb_s = bp_ref[s]
    iota_r = jax.lax.broadcasted_iota(jnp.int32, (BLK, 1), 0).astype(jnp.float32)

    for kk in range(MAXBLK):
        @pl.when(kk < nb_s)
        def _():
            row0 = ((base + kk) * BLK).astype(jnp.float32)
            g = (dest == (iota_r + row0)).astype(jnp.float32)
            xb = jnp.dot(g, x, preferred_element_type=jnp.float32)
            h = jnp.maximum(
                jnp.dot(xb, w1, preferred_element_type=jnp.float32) + b1v, 0.0
            )
            o = jnp.dot(h, w2, preferred_element_type=jnp.float32) + b2v
            norm = jnp.sqrt(jnp.sum(o * o, axis=1, keepdims=True))
            on = o / jnp.maximum(norm, 1e-12)
            contrib = jax.lax.dot_general(
                g, on, (((0,), (0,)), ((), ())),
                preferred_element_type=jnp.float32,
            )
            out_ref[...] += contrib


def kernel(eeg_emb, subject_ids, W1, b1, W2, b2):
    B, eeg_dim = eeg_emb.shape
    S, _, clip_dim = W1.shape
    sid = subject_ids.astype(jnp.int32)
    sid_row = sid.reshape(1, B)
    b1r = b1.reshape(S, 1, clip_dim)
    b2r = b2.reshape(S, 1, clip_dim)

    grid_spec = pltpu.PrefetchScalarGridSpec(
        num_scalar_prefetch=1,
        grid=(S,),
        in_specs=[
            pl.BlockSpec((1, B), lambda s, sid_ref: (0, 0)),
            pl.BlockSpec((B, eeg_dim), lambda s, sid_ref: (0, 0)),
            pl.BlockSpec((1, eeg_dim, clip_dim), lambda s, sid_ref: (s, 0, 0)),
            pl.BlockSpec((1, 1, clip_dim), lambda s, sid_ref: (s, 0, 0)),
            pl.BlockSpec((1, clip_dim, clip_dim), lambda s, sid_ref: (s, 0, 0)),
            pl.BlockSpec((1, 1, clip_dim), lambda s, sid_ref: (s, 0, 0)),
        ],
        out_specs=pl.BlockSpec((B, clip_dim), lambda s, sid_ref: (0, 0)),
        scratch_shapes=[
            pltpu.VMEM((1, B), jnp.float32),
            pltpu.SMEM((16,), jnp.int32),
            pltpu.SMEM((16,), jnp.int32),
            pltpu.SMEM((16,), jnp.int32),
        ],
    )
    out = pl.pallas_call(
        _body,
        grid_spec=grid_spec,
        out_shape=jax.ShapeDtypeStruct((B, clip_dim), jnp.float32),
    )(sid, sid_row, eeg_emb, W1, b1r, W2, b2r)
    return out


# speedup vs baseline: 1.5377x; 1.5377x over previous
import jax
import jax.numpy as jnp
from jax.experimental import pallas as pl
from jax.experimental.pallas import tpu as pltpu


def _dense_body(sid_ref, x_ref, w1_ref, b1_ref, w2_ref, b2_ref, out_ref):
    s = pl.program_id(0)
    num_s = pl.num_programs(0)

    h = jnp.maximum(
        jnp.dot(x_ref[...], w1_ref[0], preferred_element_type=jnp.float32)
        + b1_ref[0],
        0.0,
    )
    o = jnp.dot(h, w2_ref[0], preferred_element_type=jnp.float32) + b2_ref[0]
    mask = sid_ref[...] == s

    @pl.when(s == 0)
    def _():
        out_ref[...] = jnp.where(mask, o, 0.0)

    @pl.when(s > 0)
    def _():
        acc = jnp.where(mask, o, out_ref[...])

        @pl.when(s == num_s - 1)
        def _():
            norm = jnp.sqrt(jnp.sum(acc * acc, axis=1, keepdims=True))
            out_ref[...] = acc / jnp.maximum(norm, 1e-12)

        @pl.when(s != num_s - 1)
        def _():
            out_ref[...] = acc


def kernel(eeg_emb, subject_ids, W1, b1, W2, b2):
    B, eeg_dim = eeg_emb.shape
    S, _, clip_dim = W1.shape
    sid = subject_ids.astype(jnp.int32).reshape(B, 1)
    b1r = b1.reshape(S, 1, clip_dim)
    b2r = b2.reshape(S, 1, clip_dim)

    out = pl.pallas_call(
        _dense_body,
        grid=(S,),
        in_specs=[
            pl.BlockSpec((B, 1), lambda s: (0, 0)),
            pl.BlockSpec((B, eeg_dim), lambda s: (0, 0)),
            pl.BlockSpec((1, eeg_dim, clip_dim), lambda s: (s, 0, 0)),
            pl.BlockSpec((1, 1, clip_dim), lambda s: (s, 0, 0)),
            pl.BlockSpec((1, clip_dim, clip_dim), lambda s: (s, 0, 0)),
            pl.BlockSpec((1, 1, clip_dim), lambda s: (s, 0, 0)),
        ],
        out_specs=pl.BlockSpec((B, clip_dim), lambda s: (0, 0)),
        out_shape=jax.ShapeDtypeStruct((B, clip_dim), jnp.float32),
    )(sid, eeg_emb, W1, b1r, W2, b2r)
    return out


# speedup vs baseline: 1.7109x; 1.1126x over previous
import jax
import jax.numpy as jnp
from jax.experimental import pallas as pl
from jax.experimental.pallas import tpu as pltpu


def _dense_body(sid_ref, x_ref, w1_ref, b1_ref, w2_ref, b2_ref, out_ref):
    s = pl.program_id(0)
    num_s = pl.num_programs(0)

    @pl.when(s == 0)
    def _():
        out_ref[...] = jnp.zeros_like(out_ref)

    h = jnp.maximum(
        jnp.dot(x_ref[...], w1_ref[0], preferred_element_type=jnp.float32)
        + b1_ref[0],
        0.0,
    )
    o = jnp.dot(h, w2_ref[0], preferred_element_type=jnp.float32) + b2_ref[0]
    mask = sid_ref[...] == s
    acc = jnp.where(mask, o, out_ref[...])

    @pl.when(s == num_s - 1)
    def _():
        norm = jnp.sqrt(jnp.sum(acc * acc, axis=1, keepdims=True))
        out_ref[...] = acc / jnp.maximum(norm, 1e-12)

    @pl.when(s != num_s - 1)
    def _():
        out_ref[...] = acc


def kernel(eeg_emb, subject_ids, W1, b1, W2, b2):
    B, eeg_dim = eeg_emb.shape
    S, _, clip_dim = W1.shape
    sid = subject_ids.astype(jnp.int32).reshape(B, 1)
    b1r = b1.reshape(S, 1, clip_dim)
    b2r = b2.reshape(S, 1, clip_dim)

    out = pl.pallas_call(
        _dense_body,
        grid=(S,),
        in_specs=[
            pl.BlockSpec((B, 1), lambda s: (0, 0)),
            pl.BlockSpec((B, eeg_dim), lambda s: (0, 0)),
            pl.BlockSpec((1, eeg_dim, clip_dim), lambda s: (s, 0, 0)),
            pl.BlockSpec((1, 1, clip_dim), lambda s: (s, 0, 0)),
            pl.BlockSpec((1, clip_dim, clip_dim), lambda s: (s, 0, 0)),
            pl.BlockSpec((1, 1, clip_dim), lambda s: (s, 0, 0)),
        ],
        out_specs=pl.BlockSpec((B, clip_dim), lambda s: (0, 0)),
        out_shape=jax.ShapeDtypeStruct((B, clip_dim), jnp.float32),
    )(sid, eeg_emb, W1, b1r, W2, b2r)
    return out
